# baseline (speedup 1.0000x reference)
import jax
import jax.numpy as jnp
from jax import lax
from jax.experimental import pallas as pl

_HIDDEN = 768
_EXPERTS = 64
_BT = 2048


def _gating_body(flag_ref, h_ref, w1_ref, b1_ref, w2t_ref, b2_ref, out_ref):
    a1 = jnp.tanh(
        jnp.dot(h_ref[...], w1_ref[...], preferred_element_type=jnp.float32)
        + b1_ref[...]
    )
    logits = (
        lax.dot_general(
            a1, w2t_ref[...], (((1,), (1,)), ((), ())),
            preferred_element_type=jnp.float32,
        )
        + b2_ref[...]
    )

    m1 = jnp.max(logits, axis=-1, keepdims=True)
    is_max = logits == m1
    m2 = jnp.max(jnp.where(is_max, -jnp.inf, logits), axis=-1, keepdims=True)

    use_dense = flag_ref[0, 0] != 0

    @pl.when(jnp.logical_not(use_dense))
    def _sparse():
        t = jnp.exp(m2 - m1)
        p2 = t / (1.0 + t)
        p1 = 1.0 - p2
        out_ref[...] = jnp.where(is_max, p1, jnp.where(logits >= m2, p2, 0.0)).T

    @pl.when(use_dense)
    def _dense():
        e = jnp.exp(logits - m1)
        out_ref[...] = (e / jnp.sum(e, axis=-1, keepdims=True)).T


@jax.jit
def _gating(h, W1, b1, W2, b2, flag):
    tokens = h.shape[0]
    grid = (tokens // _BT,)
    out_t = pl.pallas_call(
        _gating_body,
        grid=grid,
        in_specs=[
            pl.BlockSpec((1, 1), lambda i: (0, 0)),
            pl.BlockSpec((_BT, _HIDDEN), lambda i: (i, 0)),
            pl.BlockSpec((_HIDDEN, _HIDDEN), lambda i: (0, 0)),
            pl.BlockSpec((1, _HIDDEN), lambda i: (0, 0)),
            pl.BlockSpec((_EXPERTS, _HIDDEN), lambda i: (0, 0)),
            pl.BlockSpec((1, _EXPERTS), lambda i: (0, 0)),
        ],
        out_specs=pl.BlockSpec((_EXPERTS, _BT), lambda i: (0, i)),
        out_shape=jax.ShapeDtypeStruct((_EXPERTS, tokens), jnp.float32),
    )(flag, h, W1, b1.reshape(1, _HIDDEN), W2.T, b2.reshape(1, _EXPERTS))
    return out_t.T


def kernel(h, W1, b1, W2, b2, epoch, top_k):
    warmup_epochs = 0
    if epoch is None or top_k is None:
        flag = jnp.ones((1, 1), jnp.float32)
    else:
        use_dense = (epoch < warmup_epochs) | (top_k <= 0)
        flag = jnp.asarray(use_dense, jnp.float32).reshape(1, 1)
    return _gating(h, W1, b1, W2, b2, flag)


# speedup vs baseline: 1.0610x; 1.0610x over previous
import jax
import jax.numpy as jnp
from jax import lax
from jax.experimental import pallas as pl

_HIDDEN = 768
_EXPERTS = 64
_BT = 4096


def _gating_body(flag_ref, h_ref, w1_ref, b1_ref, w2t_ref, b2_ref, out_ref):
    a1 = jnp.tanh(
        jnp.dot(h_ref[...], w1_ref[...], preferred_element_type=jnp.float32)
        + b1_ref[...]
    )
    logits = (
        lax.dot_general(
            a1, w2t_ref[...], (((1,), (1,)), ((), ())),
            preferred_element_type=jnp.float32,
        )
        + b2_ref[...]
    )

    m1 = jnp.max(logits, axis=-1, keepdims=True)
    is_max = logits == m1
    m2 = jnp.max(jnp.where(is_max, -jnp.inf, logits), axis=-1, keepdims=True)

    use_dense = flag_ref[0, 0] != 0

    @pl.when(jnp.logical_not(use_dense))
    def _sparse():
        t = jnp.exp(m2 - m1)
        p2 = t / (1.0 + t)
        p1 = 1.0 - p2
        out_ref[...] = jnp.where(is_max, p1, jnp.where(logits >= m2, p2, 0.0)).T

    @pl.when(use_dense)
    def _dense():
        e = jnp.exp(logits - m1)
        out_ref[...] = (e / jnp.sum(e, axis=-1, keepdims=True)).T


@jax.jit
def _gating(h, W1, b1, W2, b2, flag):
    tokens = h.shape[0]
    grid = (tokens // _BT,)
    out_t = pl.pallas_call(
        _gating_body,
        grid=grid,
        in_specs=[
            pl.BlockSpec((1, 1), lambda i: (0, 0)),
            pl.BlockSpec((_BT, _HIDDEN), lambda i: (i, 0)),
            pl.BlockSpec((_HIDDEN, _HIDDEN), lambda i: (0, 0)),
            pl.BlockSpec((_HIDDEN,), lambda i: (0,)),
            pl.BlockSpec((_EXPERTS, _HIDDEN), lambda i: (0, 0)),
            pl.BlockSpec((1, _EXPERTS), lambda i: (0, 0)),
        ],
        out_specs=pl.BlockSpec((_EXPERTS, _BT), lambda i: (0, i)),
        out_shape=jax.ShapeDtypeStruct((_EXPERTS, tokens), jnp.float32),
    )(flag, h, W1, b1, W2.T, b2.reshape(1, _EXPERTS))
    return out_t.T


def kernel(h, W1, b1, W2, b2, epoch, top_k):
    warmup_epochs = 0
    if epoch is None or top_k is None:
        flag = jnp.ones((1, 1), jnp.float32)
    else:
        use_dense = (epoch < warmup_epochs) | (top_k <= 0)
        flag = jnp.asarray(use_dense, jnp.float32).reshape(1, 1)
    return _gating(h, W1, b1, W2, b2, flag)
